# baseline (speedup 1.0000x reference)
import functools

import jax
import jax.numpy as jnp
from jax import lax
from jax.experimental import pallas as pl
from jax.experimental.pallas import tpu as pltpu
from jax.experimental.pallas import tpu_sc as plsc

BATCH = 4096
A_DIM = 8192
B_DIM = 4096
NUM_WORKERS = 32
ROWS = BATCH // NUM_WORKERS
SUB = 8
NSUB = ROWS // SUB
WIN = B_DIM + 128


def _slice_assign(a_hbm, b_hbm, i_hbm, out_hbm, i_v, win_buf, buf_b, shared,
                  sem_a, sem_b, sem_w, sem_s):
    wid = lax.axis_index("s") * 2 + lax.axis_index("c")
    sl = lax.axis_index("s")
    r0 = wid * ROWS

    pltpu.sync_copy(i_hbm, i_v)
    i_sc = jnp.max(i_v[...])
    lanes = lax.iota(jnp.int32, 16)
    q = i_sc >> 7
    r = i_sc & 127
    spmem = shared.at[sl]

    a_chunks = []
    for k in range(4, -1, -1):
        w = 1 << k
        mask_hi = (~(2 * w - 1)) & 31
        off1 = 128 * (q & mask_hi)
        a_chunks.append(((q & w) != 0, off1, off1, 128 * w))
        w3 = 31 - q
        off3 = 128 * (q + 33 + (w3 & mask_hi))
        a_chunks.append(((w3 & w) != 0, off3, off3 - 128 * 33, 128 * w))

    def spmem_copies(sub, direction, op):
        rows8 = pl.ds(r0 + sub * SUB, SUB)
        for cond, hoff, soff, width in a_chunks:
            def run(hoff=hoff, soff=soff, width=width):
                if direction == "in":
                    c = pltpu.make_async_copy(
                        a_hbm.at[rows8, pl.ds(hoff, width)],
                        spmem.at[:, pl.ds(soff, width)], sem_a)
                else:
                    c = pltpu.make_async_copy(
                        spmem.at[:, pl.ds(soff, width)],
                        out_hbm.at[rows8, pl.ds(hoff, width)], sem_s)
                c.start() if op == "start" else c.wait()
            pl.when(cond)(run)

    def edge_copies(sub, op):
        rows8 = pl.ds(r0 + sub * SUB, SUB)
        for woff, boff in ((0, 0), (B_DIM, 32 * 128)):
            c = pltpu.make_async_copy(
                a_hbm.at[rows8, pl.ds(128 * q + boff, 128)],
                win_buf.at[:, woff:woff + 128], sem_a)
            c.start() if op == "start" else c.wait()

    def b_copy(sub):
        rows8 = pl.ds(r0 + sub * SUB, SUB)
        return pltpu.make_async_copy(b_hbm.at[rows8, :], buf_b, sem_b)

    def win_out(sub):
        rows8 = pl.ds(r0 + sub * SUB, SUB)
        return pltpu.make_async_copy(
            win_buf, out_hbm.at[rows8, pl.ds(128 * q, WIN)], sem_w)

    def body(sub, carry):
        b_copy(sub).start()
        @pl.when(sub > 0)
        def _():
            spmem_copies(sub - 1, "out", "wait")
        spmem_copies(sub, "in", "start")
        @pl.when(sub > 0)
        def _():
            win_out(sub - 1).wait()
        edge_copies(sub, "start")
        spmem_copies(sub, "in", "wait")
        spmem_copies(sub, "out", "start")
        edge_copies(sub, "wait")
        b_copy(sub).wait()
        for row in range(SUB):
            row_v = jnp.full((16,), row, jnp.int32)
            @plsc.parallel_loop(0, B_DIM, step=16, unroll=8)
            def _overwrite(tb):
                vals = buf_b[row, pl.ds(tb, 16)]
                idx = lanes + (r + tb)
                plsc.store_scatter(win_buf, [row_v, idx], vals)
        win_out(sub).start()
        return carry

    lax.fori_loop(0, NSUB, body, 0)
    win_out(NSUB - 1).wait()
    spmem_copies(NSUB - 1, "out", "wait")


def kernel(a, b, i):
    i16 = jnp.broadcast_to(i.astype(jnp.int32), (16,))
    mesh = plsc.VectorSubcoreMesh(core_axis_name="c", subcore_axis_name="s")
    run = functools.partial(
        pl.kernel,
        mesh=mesh,
        out_type=jax.ShapeDtypeStruct((BATCH, A_DIM), jnp.float32),
        scratch_types=[
            pltpu.VMEM((16,), jnp.int32),
            pltpu.VMEM((SUB, WIN), jnp.float32),
            pltpu.VMEM((SUB, B_DIM), jnp.float32),
            pltpu.VMEM_SHARED((16, SUB, B_DIM), jnp.float32),
            pltpu.SemaphoreType.DMA,
            pltpu.SemaphoreType.DMA,
            pltpu.SemaphoreType.DMA,
            pltpu.SemaphoreType.DMA,
        ],
        compiler_params=pltpu.CompilerParams(needs_layout_passes=False),
    )(_slice_assign)
    return run(a, b, i16)
